# baseline (speedup 1.0000x reference)
import functools

import jax
import jax.numpy as jnp
from jax import lax
from jax.experimental import pallas as pl
from jax.experimental.pallas import tpu as pltpu
from jax.experimental.pallas import tpu_sc as plsc

EMB_DIM = 32
NC = 2
NS = 16
NW = NC * NS
PER_W = 25600
GSZ = 128
CH = 640
NG = CH // GSZ
G = PER_W // CH
NBUF = 4
IDX_ROWS = PER_W // GSZ
N_TOTAL = NW * PER_W

_mesh = plsc.VectorSubcoreMesh(core_axis_name="c", subcore_axis_name="s")


@functools.partial(
    pl.kernel,
    out_type=jax.ShapeDtypeStruct((N_TOTAL, EMB_DIM), jnp.float32),
    mesh=_mesh,
    scratch_types=[
        pltpu.VMEM((IDX_ROWS, GSZ), jnp.int32),
        [pltpu.VMEM((CH, EMB_DIM), jnp.float32) for _ in range(NBUF)],
        [pltpu.SemaphoreType.DMA for _ in range(NBUF)],
        [pltpu.SemaphoreType.DMA for _ in range(NBUF)],
    ],
    compiler_params=pltpu.CompilerParams(use_tc_tiling_on_sc=False),
)
def _sc_gather(idx_hbm, table_hbm, out_hbm, idx_v, bufs, gsems, wsems):
    wid = lax.axis_index("s") * NC + lax.axis_index("c")
    pltpu.sync_copy(idx_hbm.at[wid], idx_v)
    out_base = wid * PER_W

    def fire(c, b):
        for j in range(NG):
            pltpu.make_async_copy(
                table_hbm.at[idx_v.at[c * NG + j]],
                bufs[b].at[pl.ds(j * GSZ, GSZ)],
                gsems[b],
            ).start()

    def drain_g(b):
        pltpu.make_async_copy(
            out_hbm.at[pl.ds(0, CH)], bufs[b], gsems[b]).wait()

    def wb(c, b):
        pltpu.make_async_copy(
            bufs[b], out_hbm.at[pl.ds(out_base + c * CH, CH)], wsems[b]
        ).start()

    def drain_w(b):
        pltpu.make_async_copy(
            bufs[b], out_hbm.at[pl.ds(0, CH)], wsems[b]).wait()

    for b in range(NBUF):
        fire(b, b)

    nit = G // NBUF

    def body(i, carry):
        c0 = i * NBUF
        for b in range(NBUF):
            drain_g(b)
            wb(c0 + b, b)

        @pl.when(i + 1 < nit)
        def _():
            for b in range(NBUF):
                drain_w(b)
                fire(c0 + NBUF + b, b)

        return carry

    lax.fori_loop(0, nit, body, 0)
    for b in range(NBUF):
        drain_w(b)


def kernel(timestep, embeddings):
    idx = timestep.reshape(-1).astype(jnp.int32)
    idx = idx.reshape(NW, IDX_ROWS, GSZ)
    out = _sc_gather(idx, embeddings)
    return out.reshape(timestep.shape + (EMB_DIM,))
